# baseline (speedup 1.0000x reference)
import functools

import jax
import jax.numpy as jnp
from jax import lax
from jax.experimental import pallas as pl
from jax.experimental.pallas import tpu as pltpu
from jax.experimental.pallas import tpu_sc as plsc

NCLS = 19
CPAD = 32
C = 96
NPIX = 224 * 224
NB = 2
NC, NS, L = 2, 16, 16
NW = NC * NS
SPAN = NPIX // NW
BLK = 224
NCHUNK = SPAN // BLK
NGRP = BLK // L
SUMW = NCLS * C
THEA = 0.5
DELTA = 1.5
_MESH = plsc.VectorSubcoreMesh(core_axis_name="c", subcore_axis_name="s")
NPAIR_G = (NCLS * NCLS + L - 1) // L
_PARAMS = pltpu.CompilerParams(
    use_tc_tiling_on_sc=False, needs_layout_passes=False
)


def _vsqrt(d2):
    d2 = jnp.maximum(d2, jnp.float32(1e-12))
    i = plsc.bitcast(d2, jnp.int32)
    x = plsc.bitcast((i >> 1) + 0x1FBD1DF5, jnp.float32)
    x = jnp.float32(0.5) * (x + d2 / x)
    x = jnp.float32(0.5) * (x + d2 / x)
    x = jnp.float32(0.5) * (x + d2 / x)
    return x


def _zero_ref(ref, nwords):
    zf = jnp.zeros((L,), jnp.float32)

    def body(i, _):
        ref[pl.ds(i * L, L)] = zf
        return 0

    lax.fori_loop(0, nwords // L, body, 0)


def _load_centers(sums_hbm, cnt_hbm, ctr, tmp, ctmp, cvec):
    _zero_ref(ctr, SUMW)
    _zero_ref(cvec, CPAD)

    def wbody(w, _):
        pltpu.sync_copy(sums_hbm.at[w], tmp)
        pltpu.sync_copy(cnt_hbm.at[w], ctmp)

        def jbody(j, _):
            ctr[pl.ds(j * L, L)] = ctr[pl.ds(j * L, L)] + tmp[pl.ds(j * L, L)]
            return 0

        lax.fori_loop(0, SUMW // L, jbody, 0)
        cvec[pl.ds(0, L)] = cvec[pl.ds(0, L)] + ctmp[pl.ds(0, L)]
        cvec[pl.ds(L, L)] = cvec[pl.ds(L, L)] + ctmp[pl.ds(L, L)]
        return 0

    lax.fori_loop(0, NW, wbody, 0)
    inv0 = jnp.float32(1.0) / jnp.maximum(cvec[pl.ds(0, L)], jnp.float32(1.0))
    inv1 = jnp.float32(1.0) / jnp.maximum(cvec[pl.ds(L, L)], jnp.float32(1.0))
    ctmp[pl.ds(0, L)] = inv0
    ctmp[pl.ds(L, L)] = inv1
    for k in range(NCLS):
        ik = plsc.load_gather(ctmp, [jnp.full((L,), k, jnp.int32)])
        for gi in range(C // L):
            s = pl.ds(k * C + gi * L, L)
            ctr[s] = ctr[s] * ik


def _pixel_chunks(pred_hbm, tgt_hbm, pbuf, tbuf, wid, grp_fn):
    def chunk_body(t, _):
        n = t // NCHUNK
        k = t % NCHUNK
        col0 = wid * SPAN + k * BLK
        pltpu.sync_copy(pred_hbm.at[pl.ds(n * C, C), pl.ds(col0, BLK)], pbuf)
        pltpu.sync_copy(tgt_hbm.at[pl.ds(n * NPIX + col0, BLK)], tbuf)

        def grp_body(g, _):
            grp_fn(g)
            return 0

        lax.fori_loop(0, NGRP, grp_body, 0)
        return 0

    lax.fori_loop(0, NB * NCHUNK, chunk_body, 0)


def _lane_reduce(acc, out_ref, nwords):
    def body(j, _):
        v = acc[pl.ds(j * L, L)]
        for l in range(1, L):
            v = v + acc[pl.ds(l * nwords + j * L, L)]
        out_ref[pl.ds(j * L, L)] = v
        return 0

    lax.fori_loop(0, nwords // L, body, 0)


@functools.partial(
    pl.kernel,
    out_type=(
        jax.ShapeDtypeStruct((NW, SUMW), jnp.float32),
        jax.ShapeDtypeStruct((NW, CPAD), jnp.float32),
    ),
    mesh=_MESH,
    compiler_params=_PARAMS,
    scratch_types=[
        pltpu.VMEM((C, BLK), jnp.float32),
        pltpu.VMEM((BLK,), jnp.int32),
        pltpu.VMEM((L * SUMW,), jnp.float32),
        pltpu.VMEM((L * CPAD,), jnp.float32),
        pltpu.VMEM((SUMW,), jnp.float32),
        pltpu.VMEM((CPAD,), jnp.float32),
    ],
)
def _pass1(pred_hbm, tgt_hbm, sums_out, cnt_out, pbuf, tbuf, acc, cacc, sred, cred):
    wid = lax.axis_index("s") * NC + lax.axis_index("c")
    lane = lax.iota(jnp.int32, L)
    _zero_ref(acc, L * SUMW)
    _zero_ref(cacc, L * CPAD)
    ones = jnp.ones((L,), jnp.float32)

    def grp(g):
        t16 = tbuf[pl.ds(g * L, L)]
        base = lane * SUMW + t16 * C
        plsc.addupdate_scatter(cacc, [lane * CPAD + t16], ones)
        for c in range(C):
            p16 = pbuf[c, pl.ds(g * L, L)]
            plsc.addupdate_scatter(acc, [base + c], p16)

    _pixel_chunks(pred_hbm, tgt_hbm, pbuf, tbuf, wid, grp)
    _lane_reduce(acc, sred, SUMW)
    _lane_reduce(cacc, cred, CPAD)
    pltpu.sync_copy(sred, sums_out.at[wid])
    pltpu.sync_copy(cred, cnt_out.at[wid])


@functools.partial(
    pl.kernel,
    out_type=(
        jax.ShapeDtypeStruct((NW, CPAD), jnp.float32),
        jax.ShapeDtypeStruct((NW, CPAD), jnp.float32),
    ),
    mesh=_MESH,
    compiler_params=_PARAMS,
    scratch_types=[
        pltpu.VMEM((C, BLK), jnp.float32),
        pltpu.VMEM((BLK,), jnp.int32),
        pltpu.VMEM((SUMW,), jnp.float32),
        pltpu.VMEM((SUMW,), jnp.float32),
        pltpu.VMEM((CPAD,), jnp.float32),
        pltpu.VMEM((CPAD,), jnp.float32),
        pltpu.VMEM((CPAD,), jnp.float32),
        pltpu.VMEM((L * CPAD,), jnp.float32),
        pltpu.VMEM((L * CPAD,), jnp.float32),
        pltpu.VMEM((CPAD,), jnp.float32),
        pltpu.VMEM((CPAD,), jnp.float32),
    ],
)
def _pass2(pred_hbm, tgt_hbm, sums_hbm, cnt_hbm, r2_out, pos_out,
           pbuf, tbuf, ctr, tmp, ctmp, cvec, cnorm, r2acc, pacc, r2red, pred_):
    wid = lax.axis_index("s") * NC + lax.axis_index("c")
    lane = lax.iota(jnp.int32, L)
    _load_centers(sums_hbm, cnt_hbm, ctr, tmp, ctmp, cvec)
    cn0 = jnp.zeros((L,), jnp.float32)
    cn1 = jnp.zeros((L,), jnp.float32)
    for k in range(NCLS):
        a = jnp.zeros((L,), jnp.float32)
        for gi in range(C // L):
            v = ctr[pl.ds(k * C + gi * L, L)]
            a = a + v * v
        s = jnp.sum(a)
        if k < L:
            cn0 = jnp.where(lane == k, s, cn0)
        else:
            cn1 = jnp.where(lane == (k - L), s, cn1)
    cnorm[pl.ds(0, L)] = cn0
    cnorm[pl.ds(L, L)] = cn1

    _zero_ref(r2acc, L * CPAD)
    _zero_ref(pacc, L * CPAD)
    ones = jnp.ones((L,), jnp.float32)
    zeros = jnp.zeros((L,), jnp.float32)

    def grp(g):
        t16 = tbuf[pl.ds(g * L, L)]
        base = t16 * C
        pn = jnp.zeros((L,), jnp.float32)
        dt = jnp.zeros((L,), jnp.float32)
        for c in range(C):
            p16 = pbuf[c, pl.ds(g * L, L)]
            ck = plsc.load_gather(ctr, [base + c])
            pn = pn + p16 * p16
            dt = dt + p16 * ck
        cn16 = plsc.load_gather(cnorm, [t16])
        d2 = pn - jnp.float32(2.0) * dt + cn16
        r = jnp.maximum(_vsqrt(d2) - jnp.float32(THEA), jnp.float32(0.0))
        sidx = lane * CPAD + t16
        plsc.addupdate_scatter(r2acc, [sidx], r * r)
        plsc.addupdate_scatter(pacc, [sidx], jnp.where(r > 0, ones, zeros))

    _pixel_chunks(pred_hbm, tgt_hbm, pbuf, tbuf, wid, grp)
    _lane_reduce(r2acc, r2red, CPAD)
    _lane_reduce(pacc, pred_, CPAD)
    pltpu.sync_copy(r2red, r2_out.at[wid])
    pltpu.sync_copy(pred_, pos_out.at[wid])


@functools.partial(
    pl.kernel,
    out_type=jax.ShapeDtypeStruct((NW, L), jnp.float32),
    mesh=_MESH,
    compiler_params=_PARAMS,
    scratch_types=[
        pltpu.VMEM((SUMW,), jnp.float32),
        pltpu.VMEM((SUMW,), jnp.float32),
        pltpu.VMEM((CPAD,), jnp.float32),
        pltpu.VMEM((CPAD,), jnp.float32),
        pltpu.VMEM((CPAD,), jnp.float32),
        pltpu.VMEM((CPAD,), jnp.float32),
        pltpu.VMEM((CPAD,), jnp.float32),
        pltpu.VMEM((L,), jnp.float32),
        pltpu.VMEM((L,), jnp.float32),
    ],
)
def _pass3(sums_hbm, cnt_hbm, r2_hbm, pos_hbm, out_hbm,
           ctr, tmp, ctmp, cvec, vald, r2t, post, ldacc, ovbuf):
    wid = lax.axis_index("s") * NC + lax.axis_index("c")
    lane = lax.iota(jnp.int32, L)
    _load_centers(sums_hbm, cnt_hbm, ctr, tmp, ctmp, cvec)

    _zero_ref(r2t, CPAD)
    _zero_ref(post, CPAD)

    def wbody(w, _):
        pltpu.sync_copy(r2_hbm.at[w], ctmp)
        r2t[pl.ds(0, L)] = r2t[pl.ds(0, L)] + ctmp[pl.ds(0, L)]
        r2t[pl.ds(L, L)] = r2t[pl.ds(L, L)] + ctmp[pl.ds(L, L)]
        pltpu.sync_copy(pos_hbm.at[w], ctmp)
        post[pl.ds(0, L)] = post[pl.ds(0, L)] + ctmp[pl.ds(0, L)]
        post[pl.ds(L, L)] = post[pl.ds(L, L)] + ctmp[pl.ds(L, L)]
        return 0

    lax.fori_loop(0, NW, wbody, 0)

    one = jnp.float32(1.0)
    zero = jnp.float32(0.0)
    onev = jnp.ones((L,), jnp.float32)
    v0 = jnp.where(cvec[pl.ds(0, L)] > jnp.float32(20.0), one, zero)
    v1 = jnp.where(cvec[pl.ds(L, L)] > jnp.float32(20.0), one, zero)
    vald[pl.ds(0, L)] = v0
    vald[pl.ds(L, L)] = v1
    nvv = jnp.full((L,), jnp.sum(v0 + v1), jnp.float32)

    lv0 = v0 * r2t[pl.ds(0, L)] / jnp.maximum(post[pl.ds(0, L)], one)
    lv1 = v1 * r2t[pl.ds(L, L)] / jnp.maximum(post[pl.ds(L, L)], one)
    loss_var = jnp.full((L,), jnp.sum(lv0 + lv1), jnp.float32) / nvv

    ldacc[pl.ds(0, L)] = jnp.zeros((L,), jnp.float32)

    def pbody(pg, _):
        p16 = pg * L + lane
        i16 = p16 // NCLS
        j16 = p16 % NCLS
        inb = p16 < NCLS * NCLS
        i16 = jnp.minimum(i16, NCLS - 1)
        a = jnp.zeros((L,), jnp.float32)
        bi = i16 * C
        bj = j16 * C
        for c in range(C):
            d = plsc.load_gather(ctr, [bi + c]) - plsc.load_gather(ctr, [bj + c])
            a = a + d * d
        dd = _vsqrt(a)
        rel = jnp.maximum(jnp.float32(2.0 * DELTA) - dd, zero)
        vi = plsc.load_gather(vald, [i16])
        vj = plsc.load_gather(vald, [j16])
        keep = jnp.where(jnp.logical_and(inb, i16 != j16), one, zero)
        plsc.addupdate(ldacc.at[pl.ds(0, L)], keep * vi * vj * rel * rel)
        return 0

    lax.fori_loop(0, NPAIR_G, pbody, 0)
    loss_dis = (jnp.full((L,), jnp.sum(ldacc[pl.ds(0, L)]), jnp.float32)
                / (nvv * (nvv - onev)))

    regv = jnp.zeros((L,), jnp.float32)
    for grp in range(2):
        a = jnp.zeros((L,), jnp.float32)
        k16 = jnp.minimum(grp * L + lane, NCLS - 1)
        bk = k16 * C
        for c in range(C):
            v = plsc.load_gather(ctr, [bk + c])
            a = a + v * v
        regv = regv + vald[pl.ds(grp * L, L)] * _vsqrt(a)
    loss_reg = jnp.full((L,), jnp.sum(regv), jnp.float32) / nvv

    total = loss_var + loss_dis + jnp.float32(0.001) * loss_reg
    ovbuf[pl.ds(0, L)] = total
    pltpu.sync_copy(ovbuf, out_hbm.at[wid])


def kernel(predict, target):
    pred2 = predict.reshape(NB * C, NPIX)
    tgt1 = target.reshape(NB * NPIX)
    sums_p, cnt_p = _pass1(pred2, tgt1)
    r2_p, pos_p = _pass2(pred2, tgt1, sums_p, cnt_p)
    out = _pass3(sums_p, cnt_p, r2_p, pos_p)
    return out[0, 0]


# speedup vs baseline: 1.2577x; 1.2577x over previous
import functools

import jax
import jax.numpy as jnp
from jax import lax
from jax.experimental import pallas as pl
from jax.experimental.pallas import tpu as pltpu
from jax.experimental.pallas import tpu_sc as plsc

NCLS = 19
CPAD = 32
C = 96
NPIX = 224 * 224
NB = 2
NC, NS, L = 2, 16, 16
NW = NC * NS
SPAN = NPIX // NW
BLK = 224
NCHUNK = SPAN // BLK
NGRP = BLK // L
SUMW = NCLS * C
THEA = 0.5
DELTA = 1.5
_MESH = plsc.VectorSubcoreMesh(core_axis_name="c", subcore_axis_name="s")
NPAIR_G = (NCLS * NCLS + L - 1) // L
_PARAMS = pltpu.CompilerParams(
    use_tc_tiling_on_sc=False, needs_layout_passes=False
)


def _vsqrt(d2):
    d2 = jnp.maximum(d2, jnp.float32(1e-12))
    i = plsc.bitcast(d2, jnp.int32)
    x = plsc.bitcast((i >> 1) + 0x1FBD1DF5, jnp.float32)
    x = jnp.float32(0.5) * (x + d2 / x)
    x = jnp.float32(0.5) * (x + d2 / x)
    x = jnp.float32(0.5) * (x + d2 / x)
    return x


def _zero_ref(ref, nwords):
    zf = jnp.zeros((L,), jnp.float32)

    def body(i, _):
        ref[pl.ds(i * L, L)] = zf
        return 0

    lax.fori_loop(0, nwords // L, body, 0)


def _load_centers(sums_hbm, cnt_hbm, wbuf, cwbuf, ctr, ctmp, cvec):
    pltpu.sync_copy(sums_hbm, wbuf)
    pltpu.sync_copy(cnt_hbm, cwbuf)

    def jbody(j, _):
        v = wbuf[0, pl.ds(j * L, L)]
        for w in range(1, NW):
            v = v + wbuf[w, pl.ds(j * L, L)]
        ctr[pl.ds(j * L, L)] = v
        return 0

    lax.fori_loop(0, SUMW // L, jbody, 0)
    for half in range(2):
        v = cwbuf[0, pl.ds(half * L, L)]
        for w in range(1, NW):
            v = v + cwbuf[w, pl.ds(half * L, L)]
        cvec[pl.ds(half * L, L)] = v
    inv0 = jnp.float32(1.0) / jnp.maximum(cvec[pl.ds(0, L)], jnp.float32(1.0))
    inv1 = jnp.float32(1.0) / jnp.maximum(cvec[pl.ds(L, L)], jnp.float32(1.0))
    ctmp[pl.ds(0, L)] = inv0
    ctmp[pl.ds(L, L)] = inv1
    for k in range(NCLS):
        ik = plsc.load_gather(ctmp, [jnp.full((L,), k, jnp.int32)])
        for gi in range(C // L):
            s = pl.ds(k * C + gi * L, L)
            ctr[s] = ctr[s] * ik


def _pixel_chunks(pred_hbm, tgt_hbm, pbufs, tbufs, sems, wid, grp_fn):
    nchunks = NB * NCHUNK

    def slices(t):
        n = t // NCHUNK
        k = t % NCHUNK
        col0 = wid * SPAN + k * BLK
        return (
            pred_hbm.at[pl.ds(n * C, C), pl.ds(col0, BLK)],
            tgt_hbm.at[pl.ds(n * NPIX + col0, BLK)],
        )

    for b in range(2):
        ps, ts = slices(jnp.int32(b))
        pltpu.async_copy(ps, pbufs.at[b], sems[b])
        pltpu.async_copy(ts, tbufs.at[b], sems[b])

    def pair_body(i, _):
        for b in range(2):
            t = i * 2 + b
            ps, ts = slices(t)
            pltpu.make_async_copy(ps, pbufs.at[b], sems[b]).wait()
            pltpu.make_async_copy(ts, tbufs.at[b], sems[b]).wait()

            def grp_body(g, _, b=b):
                grp_fn(pbufs.at[b], tbufs.at[b], g)
                return 0

            lax.fori_loop(0, NGRP, grp_body, 0)

            @pl.when(t + 2 < nchunks)
            def _(b=b, t=t):
                ps2, ts2 = slices(t + 2)
                pltpu.async_copy(ps2, pbufs.at[b], sems[b])
                pltpu.async_copy(ts2, tbufs.at[b], sems[b])

        return 0

    lax.fori_loop(0, nchunks // 2, pair_body, 0)


def _lane_reduce(acc, out_ref, nwords):
    def body(j, _):
        v = acc[pl.ds(j * L, L)]
        for l in range(1, L):
            v = v + acc[pl.ds(l * nwords + j * L, L)]
        out_ref[pl.ds(j * L, L)] = v
        return 0

    lax.fori_loop(0, nwords // L, body, 0)


@functools.partial(
    pl.kernel,
    out_type=(
        jax.ShapeDtypeStruct((NW, SUMW), jnp.float32),
        jax.ShapeDtypeStruct((NW, CPAD), jnp.float32),
    ),
    mesh=_MESH,
    compiler_params=_PARAMS,
    scratch_types=[
        pltpu.VMEM((2, C, BLK), jnp.float32),
        pltpu.VMEM((2, BLK), jnp.int32),
        pltpu.SemaphoreType.DMA,
        pltpu.SemaphoreType.DMA,
        pltpu.VMEM((L * SUMW,), jnp.float32),
        pltpu.VMEM((L * CPAD,), jnp.float32),
        pltpu.VMEM((SUMW,), jnp.float32),
        pltpu.VMEM((CPAD,), jnp.float32),
    ],
)
def _pass1(pred_hbm, tgt_hbm, sums_out, cnt_out,
           pbufs, tbufs, sem0, sem1, acc, cacc, sred, cred):
    wid = lax.axis_index("s") * NC + lax.axis_index("c")
    lane = lax.iota(jnp.int32, L)
    _zero_ref(acc, L * SUMW)
    _zero_ref(cacc, L * CPAD)
    ones = jnp.ones((L,), jnp.float32)

    def grp(pbuf, tbuf, g):
        t16 = tbuf[pl.ds(g * L, L)]
        base = lane * SUMW + t16 * C
        plsc.addupdate_scatter(cacc, [lane * CPAD + t16], ones)
        for c in range(C):
            p16 = pbuf[c, pl.ds(g * L, L)]
            plsc.addupdate_scatter(acc, [base + c], p16)

    _pixel_chunks(pred_hbm, tgt_hbm, pbufs, tbufs, (sem0, sem1), wid, grp)
    _lane_reduce(acc, sred, SUMW)
    _lane_reduce(cacc, cred, CPAD)
    pltpu.sync_copy(sred, sums_out.at[wid])
    pltpu.sync_copy(cred, cnt_out.at[wid])


@functools.partial(
    pl.kernel,
    out_type=(
        jax.ShapeDtypeStruct((NW, CPAD), jnp.float32),
        jax.ShapeDtypeStruct((NW, CPAD), jnp.float32),
    ),
    mesh=_MESH,
    compiler_params=_PARAMS,
    scratch_types=[
        pltpu.VMEM((2, C, BLK), jnp.float32),
        pltpu.VMEM((2, BLK), jnp.int32),
        pltpu.SemaphoreType.DMA,
        pltpu.SemaphoreType.DMA,
        pltpu.VMEM((NW, SUMW), jnp.float32),
        pltpu.VMEM((NW, CPAD), jnp.float32),
        pltpu.VMEM((SUMW,), jnp.float32),
        pltpu.VMEM((CPAD,), jnp.float32),
        pltpu.VMEM((CPAD,), jnp.float32),
        pltpu.VMEM((CPAD,), jnp.float32),
        pltpu.VMEM((L * CPAD,), jnp.float32),
        pltpu.VMEM((L * CPAD,), jnp.float32),
        pltpu.VMEM((CPAD,), jnp.float32),
        pltpu.VMEM((CPAD,), jnp.float32),
    ],
)
def _pass2(pred_hbm, tgt_hbm, sums_hbm, cnt_hbm, r2_out, pos_out,
           pbufs, tbufs, sem0, sem1, wbuf, cwbuf, ctr, ctmp, cvec, cnorm,
           r2acc, pacc, r2red, pred_):
    wid = lax.axis_index("s") * NC + lax.axis_index("c")
    lane = lax.iota(jnp.int32, L)
    _load_centers(sums_hbm, cnt_hbm, wbuf, cwbuf, ctr, ctmp, cvec)
    cn0 = jnp.zeros((L,), jnp.float32)
    cn1 = jnp.zeros((L,), jnp.float32)
    for k in range(NCLS):
        a = jnp.zeros((L,), jnp.float32)
        for gi in range(C // L):
            v = ctr[pl.ds(k * C + gi * L, L)]
            a = a + v * v
        s = jnp.sum(a)
        if k < L:
            cn0 = jnp.where(lane == k, s, cn0)
        else:
            cn1 = jnp.where(lane == (k - L), s, cn1)
    cnorm[pl.ds(0, L)] = cn0
    cnorm[pl.ds(L, L)] = cn1

    _zero_ref(r2acc, L * CPAD)
    _zero_ref(pacc, L * CPAD)
    ones = jnp.ones((L,), jnp.float32)
    zeros = jnp.zeros((L,), jnp.float32)

    def grp(pbuf, tbuf, g):
        t16 = tbuf[pl.ds(g * L, L)]
        base = t16 * C
        pn = jnp.zeros((L,), jnp.float32)
        dt = jnp.zeros((L,), jnp.float32)
        for c in range(C):
            p16 = pbuf[c, pl.ds(g * L, L)]
            ck = plsc.load_gather(ctr, [base + c])
            pn = pn + p16 * p16
            dt = dt + p16 * ck
        cn16 = plsc.load_gather(cnorm, [t16])
        d2 = pn - jnp.float32(2.0) * dt + cn16
        r = jnp.maximum(_vsqrt(d2) - jnp.float32(THEA), jnp.float32(0.0))
        sidx = lane * CPAD + t16
        plsc.addupdate_scatter(r2acc, [sidx], r * r)
        plsc.addupdate_scatter(pacc, [sidx], jnp.where(r > 0, ones, zeros))

    _pixel_chunks(pred_hbm, tgt_hbm, pbufs, tbufs, (sem0, sem1), wid, grp)
    _lane_reduce(r2acc, r2red, CPAD)
    _lane_reduce(pacc, pred_, CPAD)
    pltpu.sync_copy(r2red, r2_out.at[wid])
    pltpu.sync_copy(pred_, pos_out.at[wid])


@functools.partial(
    pl.kernel,
    out_type=jax.ShapeDtypeStruct((NW, L), jnp.float32),
    mesh=_MESH,
    compiler_params=_PARAMS,
    scratch_types=[
        pltpu.VMEM((NW, SUMW), jnp.float32),
        pltpu.VMEM((NW, CPAD), jnp.float32),
        pltpu.VMEM((NW, CPAD), jnp.float32),
        pltpu.VMEM((NW, CPAD), jnp.float32),
        pltpu.VMEM((SUMW,), jnp.float32),
        pltpu.VMEM((CPAD,), jnp.float32),
        pltpu.VMEM((CPAD,), jnp.float32),
        pltpu.VMEM((CPAD,), jnp.float32),
        pltpu.VMEM((L,), jnp.float32),
        pltpu.VMEM((L,), jnp.float32),
    ],
)
def _pass3(sums_hbm, cnt_hbm, r2_hbm, pos_hbm, out_hbm,
           wbuf, cwbuf, r2wbuf, poswbuf, ctr, ctmp, cvec, vald, ldacc, ovbuf):
    wid = lax.axis_index("s") * NC + lax.axis_index("c")
    lane = lax.iota(jnp.int32, L)
    _load_centers(sums_hbm, cnt_hbm, wbuf, cwbuf, ctr, ctmp, cvec)

    pltpu.sync_copy(r2_hbm, r2wbuf)
    pltpu.sync_copy(pos_hbm, poswbuf)
    r2t0 = r2wbuf[0, pl.ds(0, L)]
    r2t1 = r2wbuf[0, pl.ds(L, L)]
    post0 = poswbuf[0, pl.ds(0, L)]
    post1 = poswbuf[0, pl.ds(L, L)]
    for w in range(1, NW):
        r2t0 = r2t0 + r2wbuf[w, pl.ds(0, L)]
        r2t1 = r2t1 + r2wbuf[w, pl.ds(L, L)]
        post0 = post0 + poswbuf[w, pl.ds(0, L)]
        post1 = post1 + poswbuf[w, pl.ds(L, L)]

    one = jnp.float32(1.0)
    zero = jnp.float32(0.0)
    onev = jnp.ones((L,), jnp.float32)
    v0 = jnp.where(cvec[pl.ds(0, L)] > jnp.float32(20.0), one, zero)
    v1 = jnp.where(cvec[pl.ds(L, L)] > jnp.float32(20.0), one, zero)
    vald[pl.ds(0, L)] = v0
    vald[pl.ds(L, L)] = v1
    nvv = jnp.full((L,), jnp.sum(v0 + v1), jnp.float32)

    lv0 = v0 * r2t0 / jnp.maximum(post0, one)
    lv1 = v1 * r2t1 / jnp.maximum(post1, one)
    loss_var = jnp.full((L,), jnp.sum(lv0 + lv1), jnp.float32) / nvv

    ldacc[pl.ds(0, L)] = jnp.zeros((L,), jnp.float32)

    def pbody(pg, _):
        p16 = pg * L + lane
        i16 = p16 // NCLS
        j16 = p16 % NCLS
        inb = p16 < NCLS * NCLS
        i16 = jnp.minimum(i16, NCLS - 1)
        a = jnp.zeros((L,), jnp.float32)
        bi = i16 * C
        bj = j16 * C
        for c in range(C):
            d = plsc.load_gather(ctr, [bi + c]) - plsc.load_gather(ctr, [bj + c])
            a = a + d * d
        dd = _vsqrt(a)
        rel = jnp.maximum(jnp.float32(2.0 * DELTA) - dd, zero)
        vi = plsc.load_gather(vald, [i16])
        vj = plsc.load_gather(vald, [j16])
        keep = jnp.where(jnp.logical_and(inb, i16 != j16), one, zero)
        plsc.addupdate(ldacc.at[pl.ds(0, L)], keep * vi * vj * rel * rel)
        return 0

    lax.fori_loop(0, NPAIR_G, pbody, 0)
    loss_dis = (jnp.full((L,), jnp.sum(ldacc[pl.ds(0, L)]), jnp.float32)
                / (nvv * (nvv - onev)))

    regv = jnp.zeros((L,), jnp.float32)
    for grp in range(2):
        a = jnp.zeros((L,), jnp.float32)
        k16 = jnp.minimum(grp * L + lane, NCLS - 1)
        bk = k16 * C
        for c in range(C):
            v = plsc.load_gather(ctr, [bk + c])
            a = a + v * v
        regv = regv + vald[pl.ds(grp * L, L)] * _vsqrt(a)
    loss_reg = jnp.full((L,), jnp.sum(regv), jnp.float32) / nvv

    total = loss_var + loss_dis + jnp.float32(0.001) * loss_reg
    ovbuf[pl.ds(0, L)] = total
    pltpu.sync_copy(ovbuf, out_hbm.at[wid])


def kernel(predict, target):
    pred2 = predict.reshape(NB * C, NPIX)
    tgt1 = target.reshape(NB * NPIX)
    sums_p, cnt_p = _pass1(pred2, tgt1)
    r2_p, pos_p = _pass2(pred2, tgt1, sums_p, cnt_p)
    out = _pass3(sums_p, cnt_p, r2_p, pos_p)
    return out[0, 0]


# speedup vs baseline: 2.2704x; 1.8052x over previous
import functools

import jax
import jax.numpy as jnp
from jax import lax
from jax.experimental import pallas as pl
from jax.experimental.pallas import tpu as pltpu
from jax.experimental.pallas import tpu_sc as plsc

NCLS = 19
CPAD = 32
C = 96
CROW = 97
SUMP = 1856
ACS = 1857
CCS = 33
NPIX = 224 * 224
NB = 2
NC, NS, L = 2, 16, 16
NW = NC * NS
WB = 8
SPAN = NPIX // NW
BLK = 224
NCHUNK = SPAN // BLK
NGRP = BLK // L
THEA = 0.5
DELTA = 1.5
_MESH = plsc.VectorSubcoreMesh(core_axis_name="c", subcore_axis_name="s")
NPAIR_G = (NCLS * NCLS + L - 1) // L
_PARAMS = pltpu.CompilerParams(
    use_tc_tiling_on_sc=False, needs_layout_passes=False
)


def _vsqrt(d2):
    d2 = jnp.maximum(d2, jnp.float32(1e-12))
    i = plsc.bitcast(d2, jnp.int32)
    x = plsc.bitcast((i >> 1) + 0x1FBD1DF5, jnp.float32)
    x = jnp.float32(0.5) * (x + d2 / x)
    x = jnp.float32(0.5) * (x + d2 / x)
    x = jnp.float32(0.5) * (x + d2 / x)
    return x


def _zero_ref(ref, nwords):
    zf = jnp.zeros((L,), jnp.float32)

    def body(i, _):
        ref[pl.ds(i * L, L)] = zf
        return 0

    lax.fori_loop(0, nwords // L, body, 0)


def _build_centers(sums_hbm, cnt_hbm, wbuf, cwbuf, ctr, ctmp, cvec, nrep):
    _zero_ref(ctr, SUMP)

    for b0 in range(0, NW, WB):
        pltpu.sync_copy(sums_hbm.at[pl.ds(b0, WB)], wbuf)

        def jbody(j, _):
            v = wbuf[0, pl.ds(j * L, L)]
            for w in range(1, WB):
                v = v + wbuf[w, pl.ds(j * L, L)]
            ctr[pl.ds(j * L, L)] = ctr[pl.ds(j * L, L)] + v
            return 0

        lax.fori_loop(0, SUMP // L, jbody, 0)

    pltpu.sync_copy(cnt_hbm, cwbuf)
    for half in range(2):
        v = cwbuf[0, pl.ds(half * L, L)]
        for w in range(1, NW):
            v = v + cwbuf[w, pl.ds(half * L, L)]
        cvec[pl.ds(half * L, L)] = v
    inv0 = jnp.float32(1.0) / jnp.maximum(cvec[pl.ds(0, L)], jnp.float32(1.0))
    inv1 = jnp.float32(1.0) / jnp.maximum(cvec[pl.ds(L, L)], jnp.float32(1.0))
    ctmp[pl.ds(0, L)] = inv0
    ctmp[pl.ds(L, L)] = inv1
    for k in range(NCLS):
        ik = plsc.load_gather(ctmp, [jnp.full((L,), k, jnp.int32)])
        for gi in range(C // L):
            v = ctr[pl.ds(k * CROW + gi * L, L)] * ik
            for rep in range(nrep):
                ctr[pl.ds(rep * ACS + k * CROW + gi * L, L)] = v


def _pixel_chunks(pred_hbm, tgt_hbm, pbufs, tbufs, sems, wid, grp_fn):
    nchunks = NB * NCHUNK

    def slices(t):
        n = t // NCHUNK
        k = t % NCHUNK
        col0 = wid * SPAN + k * BLK
        return (
            pred_hbm.at[pl.ds(n * C, C), pl.ds(col0, BLK)],
            tgt_hbm.at[pl.ds(n * NPIX + col0, BLK)],
        )

    for b in range(2):
        ps, ts = slices(jnp.int32(b))
        pltpu.async_copy(ps, pbufs.at[b], sems[b])
        pltpu.async_copy(ts, tbufs.at[b], sems[b])

    def pair_body(i, _):
        for b in range(2):
            t = i * 2 + b
            ps, ts = slices(t)
            pltpu.make_async_copy(ps, pbufs.at[b], sems[b]).wait()
            pltpu.make_async_copy(ts, tbufs.at[b], sems[b]).wait()

            def grp_body(g, _, b=b):
                grp_fn(pbufs.at[b], tbufs.at[b], g)
                return 0

            lax.fori_loop(0, NGRP, grp_body, 0)

            @pl.when(t + 2 < nchunks)
            def _(b=b, t=t):
                ps2, ts2 = slices(t + 2)
                pltpu.async_copy(ps2, pbufs.at[b], sems[b])
                pltpu.async_copy(ts2, tbufs.at[b], sems[b])

        return 0

    lax.fori_loop(0, nchunks // 2, pair_body, 0)


def _lane_reduce(acc, out_ref, nwords, lstride):
    def body(j, _):
        v = acc[pl.ds(j * L, L)]
        for l in range(1, L):
            v = v + acc[pl.ds(l * lstride + j * L, L)]
        out_ref[pl.ds(j * L, L)] = v
        return 0

    lax.fori_loop(0, nwords // L, body, 0)


@functools.partial(
    pl.kernel,
    out_type=(
        jax.ShapeDtypeStruct((NW, SUMP), jnp.float32),
        jax.ShapeDtypeStruct((NW, CPAD), jnp.float32),
    ),
    mesh=_MESH,
    compiler_params=_PARAMS,
    scratch_types=[
        pltpu.VMEM((2, C, BLK), jnp.float32),
        pltpu.VMEM((2, BLK), jnp.int32),
        pltpu.SemaphoreType.DMA,
        pltpu.SemaphoreType.DMA,
        pltpu.VMEM((L * ACS,), jnp.float32),
        pltpu.VMEM((L * CCS,), jnp.float32),
        pltpu.VMEM((SUMP,), jnp.float32),
        pltpu.VMEM((CPAD,), jnp.float32),
    ],
)
def _pass1(pred_hbm, tgt_hbm, sums_out, cnt_out,
           pbufs, tbufs, sem0, sem1, acc, cacc, sred, cred):
    wid = lax.axis_index("s") * NC + lax.axis_index("c")
    lane = lax.iota(jnp.int32, L)
    _zero_ref(acc, L * ACS)
    _zero_ref(cacc, L * CCS)
    ones = jnp.ones((L,), jnp.float32)

    def grp(pbuf, tbuf, g):
        t16 = tbuf[pl.ds(g * L, L)]
        base = lane * ACS + t16 * CROW
        plsc.addupdate_scatter(cacc, [lane * CCS + t16], ones)
        for c in range(C):
            p16 = pbuf[c, pl.ds(g * L, L)]
            plsc.addupdate_scatter(acc, [base + c], p16)

    _pixel_chunks(pred_hbm, tgt_hbm, pbufs, tbufs, (sem0, sem1), wid, grp)
    _lane_reduce(acc, sred, SUMP, ACS)
    _lane_reduce(cacc, cred, CPAD, CCS)
    pltpu.sync_copy(sred, sums_out.at[wid])
    pltpu.sync_copy(cred, cnt_out.at[wid])


@functools.partial(
    pl.kernel,
    out_type=(
        jax.ShapeDtypeStruct((NW, CPAD), jnp.float32),
        jax.ShapeDtypeStruct((NW, CPAD), jnp.float32),
    ),
    mesh=_MESH,
    compiler_params=_PARAMS,
    scratch_types=[
        pltpu.VMEM((2, C, BLK), jnp.float32),
        pltpu.VMEM((2, BLK), jnp.int32),
        pltpu.SemaphoreType.DMA,
        pltpu.SemaphoreType.DMA,
        pltpu.VMEM((WB, SUMP), jnp.float32),
        pltpu.VMEM((NW, CPAD), jnp.float32),
        pltpu.VMEM((L * ACS,), jnp.float32),
        pltpu.VMEM((CPAD,), jnp.float32),
        pltpu.VMEM((CPAD,), jnp.float32),
        pltpu.VMEM((L * CCS,), jnp.float32),
        pltpu.VMEM((L * CCS,), jnp.float32),
        pltpu.VMEM((L * CCS,), jnp.float32),
        pltpu.VMEM((CPAD,), jnp.float32),
        pltpu.VMEM((CPAD,), jnp.float32),
    ],
)
def _pass2(pred_hbm, tgt_hbm, sums_hbm, cnt_hbm, r2_out, pos_out,
           pbufs, tbufs, sem0, sem1, wbuf, cwbuf, ctr, ctmp, cvec, cnorm,
           r2acc, pacc, r2red, pred_):
    wid = lax.axis_index("s") * NC + lax.axis_index("c")
    lane = lax.iota(jnp.int32, L)
    _build_centers(sums_hbm, cnt_hbm, wbuf, cwbuf, ctr, ctmp, cvec, nrep=L)
    cn0 = jnp.zeros((L,), jnp.float32)
    cn1 = jnp.zeros((L,), jnp.float32)
    for k in range(NCLS):
        a = jnp.zeros((L,), jnp.float32)
        for gi in range(C // L):
            v = ctr[pl.ds(k * CROW + gi * L, L)]
            a = a + v * v
        s = jnp.sum(a)
        if k < L:
            cn0 = jnp.where(lane == k, s, cn0)
        else:
            cn1 = jnp.where(lane == (k - L), s, cn1)
    for rep in range(L):
        cnorm[pl.ds(rep * CCS, L)] = cn0
        cnorm[pl.ds(rep * CCS + L, L)] = cn1

    _zero_ref(r2acc, L * CCS)
    _zero_ref(pacc, L * CCS)
    ones = jnp.ones((L,), jnp.float32)
    zeros = jnp.zeros((L,), jnp.float32)

    def grp(pbuf, tbuf, g):
        t16 = tbuf[pl.ds(g * L, L)]
        base = lane * ACS + t16 * CROW
        pn = jnp.zeros((L,), jnp.float32)
        dt = jnp.zeros((L,), jnp.float32)
        for c in range(C):
            p16 = pbuf[c, pl.ds(g * L, L)]
            ck = plsc.load_gather(ctr, [base + c])
            pn = pn + p16 * p16
            dt = dt + p16 * ck
        cn16 = plsc.load_gather(cnorm, [lane * CCS + t16])
        d2 = pn - jnp.float32(2.0) * dt + cn16
        r = jnp.maximum(_vsqrt(d2) - jnp.float32(THEA), jnp.float32(0.0))
        sidx = lane * CCS + t16
        plsc.addupdate_scatter(r2acc, [sidx], r * r)
        plsc.addupdate_scatter(pacc, [sidx], jnp.where(r > 0, ones, zeros))

    _pixel_chunks(pred_hbm, tgt_hbm, pbufs, tbufs, (sem0, sem1), wid, grp)
    _lane_reduce(r2acc, r2red, CPAD, CCS)
    _lane_reduce(pacc, pred_, CPAD, CCS)
    pltpu.sync_copy(r2red, r2_out.at[wid])
    pltpu.sync_copy(pred_, pos_out.at[wid])


@functools.partial(
    pl.kernel,
    out_type=jax.ShapeDtypeStruct((NW, L), jnp.float32),
    mesh=_MESH,
    compiler_params=_PARAMS,
    scratch_types=[
        pltpu.VMEM((WB, SUMP), jnp.float32),
        pltpu.VMEM((NW, CPAD), jnp.float32),
        pltpu.VMEM((NW, CPAD), jnp.float32),
        pltpu.VMEM((NW, CPAD), jnp.float32),
        pltpu.VMEM((SUMP,), jnp.float32),
        pltpu.VMEM((CPAD,), jnp.float32),
        pltpu.VMEM((CPAD,), jnp.float32),
        pltpu.VMEM((CPAD,), jnp.float32),
        pltpu.VMEM((L,), jnp.float32),
        pltpu.VMEM((L,), jnp.float32),
    ],
)
def _pass3(sums_hbm, cnt_hbm, r2_hbm, pos_hbm, out_hbm,
           wbuf, cwbuf, r2wbuf, poswbuf, ctr, ctmp, cvec, vald, ldacc, ovbuf):
    wid = lax.axis_index("s") * NC + lax.axis_index("c")
    lane = lax.iota(jnp.int32, L)
    _build_centers(sums_hbm, cnt_hbm, wbuf, cwbuf, ctr, ctmp, cvec, nrep=1)

    pltpu.sync_copy(r2_hbm, r2wbuf)
    pltpu.sync_copy(pos_hbm, poswbuf)
    r2t0 = r2wbuf[0, pl.ds(0, L)]
    r2t1 = r2wbuf[0, pl.ds(L, L)]
    post0 = poswbuf[0, pl.ds(0, L)]
    post1 = poswbuf[0, pl.ds(L, L)]
    for w in range(1, NW):
        r2t0 = r2t0 + r2wbuf[w, pl.ds(0, L)]
        r2t1 = r2t1 + r2wbuf[w, pl.ds(L, L)]
        post0 = post0 + poswbuf[w, pl.ds(0, L)]
        post1 = post1 + poswbuf[w, pl.ds(L, L)]

    one = jnp.float32(1.0)
    zero = jnp.float32(0.0)
    onev = jnp.ones((L,), jnp.float32)
    v0 = jnp.where(cvec[pl.ds(0, L)] > jnp.float32(20.0), one, zero)
    v1 = jnp.where(cvec[pl.ds(L, L)] > jnp.float32(20.0), one, zero)
    vald[pl.ds(0, L)] = v0
    vald[pl.ds(L, L)] = v1
    nvv = jnp.full((L,), jnp.sum(v0 + v1), jnp.float32)

    lv0 = v0 * r2t0 / jnp.maximum(post0, one)
    lv1 = v1 * r2t1 / jnp.maximum(post1, one)
    loss_var = jnp.full((L,), jnp.sum(lv0 + lv1), jnp.float32) / nvv

    ldacc[pl.ds(0, L)] = jnp.zeros((L,), jnp.float32)

    def pbody(pg, _):
        p16 = pg * L + lane
        i16 = p16 // NCLS
        j16 = p16 % NCLS
        inb = p16 < NCLS * NCLS
        i16 = jnp.minimum(i16, NCLS - 1)
        a = jnp.zeros((L,), jnp.float32)
        bi = i16 * CROW
        bj = j16 * CROW
        for c in range(C):
            d = plsc.load_gather(ctr, [bi + c]) - plsc.load_gather(ctr, [bj + c])
            a = a + d * d
        dd = _vsqrt(a)
        rel = jnp.maximum(jnp.float32(2.0 * DELTA) - dd, zero)
        vi = plsc.load_gather(vald, [i16])
        vj = plsc.load_gather(vald, [j16])
        keep = jnp.where(jnp.logical_and(inb, i16 != j16), one, zero)
        plsc.addupdate(ldacc.at[pl.ds(0, L)], keep * vi * vj * rel * rel)
        return 0

    lax.fori_loop(0, NPAIR_G, pbody, 0)
    loss_dis = (jnp.full((L,), jnp.sum(ldacc[pl.ds(0, L)]), jnp.float32)
                / (nvv * (nvv - onev)))

    regv = jnp.zeros((L,), jnp.float32)
    for grp in range(2):
        a = jnp.zeros((L,), jnp.float32)
        k16 = jnp.minimum(grp * L + lane, NCLS - 1)
        bk = k16 * CROW
        for c in range(C):
            v = plsc.load_gather(ctr, [bk + c])
            a = a + v * v
        regv = regv + vald[pl.ds(grp * L, L)] * _vsqrt(a)
    loss_reg = jnp.full((L,), jnp.sum(regv), jnp.float32) / nvv

    total = loss_var + loss_dis + jnp.float32(0.001) * loss_reg
    ovbuf[pl.ds(0, L)] = total
    pltpu.sync_copy(ovbuf, out_hbm.at[wid])


def kernel(predict, target):
    pred2 = predict.reshape(NB * C, NPIX)
    tgt1 = target.reshape(NB * NPIX)
    sums_p, cnt_p = _pass1(pred2, tgt1)
    r2_p, pos_p = _pass2(pred2, tgt1, sums_p, cnt_p)
    out = _pass3(sums_p, cnt_p, r2_p, pos_p)
    return out[0, 0]
